# Initial kernel scaffold; baseline (speedup 1.0000x reference)
import jax
import jax.numpy as jnp
from jax.experimental import pallas as pl


def kernel(primals_1, primals_2, primals_3, primals_4, primals_5, primals_6, primals_7, primals_8):
    raise NotImplementedError("write your pallas kernel here")


# speedup vs baseline: 1.4370x; 1.4370x over previous
import functools

import jax
import jax.numpy as jnp
from jax import lax
from jax.experimental import pallas as pl
from jax.experimental.pallas import tpu as pltpu
import jax.experimental.pallas.tpu_sc as plsc

N = 262144
NC = 2
NS = 16
NW = NC * NS
GCHUNK = N // NW
DEST = 6 * 256 * 256
OWN = DEST // NW
SCHUNK = 16384
NSCHUNK = N // SCHUNK

_mesh = plsc.VectorSubcoreMesh(
    core_axis_name="c", subcore_axis_name="s", num_cores=NC, num_subcores=NS
)
_sc_params = pltpu.CompilerParams(needs_layout_passes=False)


def _wid():
    return lax.axis_index("s") * NC + lax.axis_index("c")


def _gather_body(p2_hbm, p3_hbm, p4_hbm, p5_hbm, p6_hbm,
                 dest_hbm, vals_hbm,
                 i3, i4, i5, i6, lin, dst, vals, sem):
    base = _wid() * GCHUNK
    pltpu.sync_copy(p3_hbm.at[pl.ds(base, GCHUNK)], i3)
    pltpu.sync_copy(p4_hbm.at[pl.ds(base, GCHUNK)], i4)
    pltpu.sync_copy(p5_hbm.at[pl.ds(base, GCHUNK)], i5)
    pltpu.sync_copy(p6_hbm.at[pl.ds(base, GCHUNK)], i6)

    def body(j, carry):
        sl = pl.ds(j * 16, 16)
        a3 = i3[sl]
        lin[sl] = a3 * 1000000 + i4[sl]
        dst[sl] = a3 * 65536 + i5[sl] * 256 + i6[sl]
        return carry

    lax.fori_loop(0, GCHUNK // 16, body, 0)
    pltpu.async_copy(p2_hbm.at[lin], vals, sem).wait()
    pltpu.sync_copy(dst, dest_hbm.at[pl.ds(base, GCHUNK)])
    pltpu.sync_copy(vals, vals_hbm.at[pl.ds(base, GCHUNK)])


_gather_call = pl.kernel(
    _gather_body,
    out_type=(
        jax.ShapeDtypeStruct((N,), jnp.int32),
        jax.ShapeDtypeStruct((N,), jnp.float32),
    ),
    mesh=_mesh,
    scratch_types=[
        pltpu.VMEM((GCHUNK,), jnp.int32),
        pltpu.VMEM((GCHUNK,), jnp.int32),
        pltpu.VMEM((GCHUNK,), jnp.int32),
        pltpu.VMEM((GCHUNK,), jnp.int32),
        pltpu.VMEM((GCHUNK,), jnp.int32),
        pltpu.VMEM((GCHUNK,), jnp.int32),
        pltpu.VMEM((GCHUNK,), jnp.float32),
        pltpu.SemaphoreType.DMA,
    ],
    compiler_params=_sc_params,
)


def _scatter_body(dest_hbm, vals_hbm, p1_hbm, out_hbm, local, dbuf, vbuf):
    wid = _wid()
    lo = wid * OWN
    pltpu.sync_copy(p1_hbm.at[pl.ds(lo, OWN)], local)
    lo_v = jnp.full((16,), 0, jnp.int32) + lo

    for c in range(NSCHUNK):
        pltpu.sync_copy(dest_hbm.at[pl.ds(c * SCHUNK, SCHUNK)], dbuf)
        pltpu.sync_copy(vals_hbm.at[pl.ds(c * SCHUNK, SCHUNK)], vbuf)

        def inner(j, carry):
            sl = pl.ds(j * 16, 16)
            u = dbuf[sl] - lo_v
            m = (u >= 0) & (u < OWN)
            plsc.store_scatter(local, [u], vbuf[sl], mask=m)
            return carry

        lax.fori_loop(0, SCHUNK // 16, inner, 0)

    pltpu.sync_copy(local, out_hbm.at[pl.ds(lo, OWN)])


_scatter_call = pl.kernel(
    _scatter_body,
    out_type=jax.ShapeDtypeStruct((DEST,), jnp.float32),
    mesh=_mesh,
    scratch_types=[
        pltpu.VMEM((OWN,), jnp.float32),
        pltpu.VMEM((SCHUNK,), jnp.int32),
        pltpu.VMEM((SCHUNK,), jnp.float32),
    ],
    compiler_params=_sc_params,
)


def _tc_body(ip_ref, p7_ref, v_ref, bmm_ref, pm6_ref):
    add = ip_ref[0] + p7_ref[0] * 0.975
    bmm_ref[0] = jnp.dot(v_ref[0], add, preferred_element_type=jnp.float32)
    pm6_ref[0] = add.T


_tc_call = pl.pallas_call(
    _tc_body,
    grid=(6,),
    in_specs=[
        pl.BlockSpec((1, 256, 256), lambda b: (b, 0, 0)),
        pl.BlockSpec((1, 256, 256), lambda b: (b, 0, 0)),
        pl.BlockSpec((1, 12, 256), lambda b: (b, 0, 0)),
    ],
    out_specs=[
        pl.BlockSpec((1, 12, 256), lambda b: (b, 0, 0)),
        pl.BlockSpec((1, 256, 256), lambda b: (b, 0, 0)),
    ],
    out_shape=[
        jax.ShapeDtypeStruct((6, 12, 256), jnp.float32),
        jax.ShapeDtypeStruct((6, 256, 256), jnp.float32),
    ],
)


@jax.jit
def kernel(primals_1, primals_2, primals_3, primals_4, primals_5, primals_6,
           primals_7, primals_8):
    p2f = primals_2.reshape(-1)
    p1f = primals_1.reshape(-1)
    dest, vals = _gather_call(p2f, primals_3, primals_4, primals_5, primals_6)
    ipf = _scatter_call(dest, vals, p1f)
    ip = ipf.reshape(6, 256, 256)
    view = jnp.transpose(primals_8, (1, 0, 2))
    bmm6, pm6 = _tc_call(ip, primals_7, view)
    view_3 = jnp.transpose(bmm6, (1, 0, 2))
    return (view_3, pm6)


# speedup vs baseline: 1.5507x; 1.0791x over previous
import functools

import jax
import jax.numpy as jnp
from jax import lax
from jax.experimental import pallas as pl
from jax.experimental.pallas import tpu as pltpu
import jax.experimental.pallas.tpu_sc as plsc

N = 262144
NC = 2
NS = 16
NW = NC * NS
GCHUNK = N // NW
DEST = 6 * 256 * 256
OWN = DEST // NW
SCHUNK = 16384
NSCHUNK = N // SCHUNK

_mesh = plsc.VectorSubcoreMesh(
    core_axis_name="c", subcore_axis_name="s", num_cores=NC, num_subcores=NS
)
_sc_params = pltpu.CompilerParams(needs_layout_passes=False)


def _wid():
    return lax.axis_index("s") * NC + lax.axis_index("c")


def _gather_body(p2_hbm, p3_hbm, p4_hbm, p5_hbm, p6_hbm,
                 dest_hbm, vals_hbm,
                 i3, i4, i5, i6, lin, dst, vals, sem):
    base = _wid() * GCHUNK
    pltpu.sync_copy(p3_hbm.at[pl.ds(base, GCHUNK)], i3)
    pltpu.sync_copy(p4_hbm.at[pl.ds(base, GCHUNK)], i4)
    pltpu.sync_copy(p5_hbm.at[pl.ds(base, GCHUNK)], i5)
    pltpu.sync_copy(p6_hbm.at[pl.ds(base, GCHUNK)], i6)

    def body(j, carry):
        sl = pl.ds(j * 16, 16)
        a3 = i3[sl]
        lin[sl] = a3 * 1000000 + i4[sl]
        dst[sl] = a3 * 65536 + i5[sl] * 256 + i6[sl]
        return carry

    lax.fori_loop(0, GCHUNK // 16, body, 0, unroll=8)
    pltpu.async_copy(p2_hbm.at[lin], vals, sem).wait()
    pltpu.sync_copy(dst, dest_hbm.at[pl.ds(base, GCHUNK)])
    pltpu.sync_copy(vals, vals_hbm.at[pl.ds(base, GCHUNK)])


_gather_call = pl.kernel(
    _gather_body,
    out_type=(
        jax.ShapeDtypeStruct((N,), jnp.int32),
        jax.ShapeDtypeStruct((N,), jnp.float32),
    ),
    mesh=_mesh,
    scratch_types=[
        pltpu.VMEM((GCHUNK,), jnp.int32),
        pltpu.VMEM((GCHUNK,), jnp.int32),
        pltpu.VMEM((GCHUNK,), jnp.int32),
        pltpu.VMEM((GCHUNK,), jnp.int32),
        pltpu.VMEM((GCHUNK,), jnp.int32),
        pltpu.VMEM((GCHUNK,), jnp.int32),
        pltpu.VMEM((GCHUNK,), jnp.float32),
        pltpu.SemaphoreType.DMA,
    ],
    compiler_params=_sc_params,
)


def _scatter_body(dest_hbm, vals_hbm, p1_hbm, out_hbm, local,
                  dbuf0, dbuf1, vbuf0, vbuf1, semd, semv):
    wid = _wid()
    lo = wid * OWN
    pltpu.sync_copy(p1_hbm.at[pl.ds(lo, OWN)], local)
    lo_v = jnp.full((16,), 0, jnp.int32) + lo
    own_v = jnp.full((16,), OWN, jnp.uint32)

    dbufs = (dbuf0, dbuf1)
    vbufs = (vbuf0, vbuf1)

    def start(c, b):
        pltpu.async_copy(dest_hbm.at[pl.ds(c * SCHUNK, SCHUNK)], dbufs[b], semd)
        pltpu.async_copy(vals_hbm.at[pl.ds(c * SCHUNK, SCHUNK)], vbufs[b], semv)

    def wait(c, b):
        pltpu.make_async_copy(dest_hbm.at[pl.ds(c * SCHUNK, SCHUNK)],
                              dbufs[b], semd).wait()
        pltpu.make_async_copy(vals_hbm.at[pl.ds(c * SCHUNK, SCHUNK)],
                              vbufs[b], semv).wait()

    start(0, 0)
    for c in range(NSCHUNK):
        b = c % 2
        wait(c, b)
        if c + 1 < NSCHUNK:
            start(c + 1, 1 - b)
        dbuf = dbufs[b]
        vbuf = vbufs[b]

        def inner(j, carry):
            sl = pl.ds(j * 16, 16)
            u = plsc.bitcast(dbuf[sl] - lo_v, jnp.uint32)
            m = u < own_v
            plsc.store_scatter(local, [plsc.bitcast(u, jnp.int32)],
                               vbuf[sl], mask=m)
            return carry

        lax.fori_loop(0, SCHUNK // 16, inner, 0, unroll=8)

    pltpu.sync_copy(local, out_hbm.at[pl.ds(lo, OWN)])


_scatter_call = pl.kernel(
    _scatter_body,
    out_type=jax.ShapeDtypeStruct((DEST,), jnp.float32),
    mesh=_mesh,
    scratch_types=[
        pltpu.VMEM((OWN,), jnp.float32),
        pltpu.VMEM((SCHUNK,), jnp.int32),
        pltpu.VMEM((SCHUNK,), jnp.int32),
        pltpu.VMEM((SCHUNK,), jnp.float32),
        pltpu.VMEM((SCHUNK,), jnp.float32),
        pltpu.SemaphoreType.DMA,
        pltpu.SemaphoreType.DMA,
    ],
    compiler_params=_sc_params,
)


def _tc_body(ip_ref, p7_ref, v_ref, bmm_ref, pm6_ref):
    add = ip_ref[0] + p7_ref[0] * 0.975
    bmm_ref[0] = jnp.dot(v_ref[0], add, preferred_element_type=jnp.float32)
    pm6_ref[0] = add.T


_tc_call = pl.pallas_call(
    _tc_body,
    grid=(6,),
    in_specs=[
        pl.BlockSpec((1, 256, 256), lambda b: (b, 0, 0)),
        pl.BlockSpec((1, 256, 256), lambda b: (b, 0, 0)),
        pl.BlockSpec((1, 12, 256), lambda b: (b, 0, 0)),
    ],
    out_specs=[
        pl.BlockSpec((1, 12, 256), lambda b: (b, 0, 0)),
        pl.BlockSpec((1, 256, 256), lambda b: (b, 0, 0)),
    ],
    out_shape=[
        jax.ShapeDtypeStruct((6, 12, 256), jnp.float32),
        jax.ShapeDtypeStruct((6, 256, 256), jnp.float32),
    ],
)


@jax.jit
def kernel(primals_1, primals_2, primals_3, primals_4, primals_5, primals_6,
           primals_7, primals_8):
    p2f = primals_2.reshape(-1)
    p1f = primals_1.reshape(-1)
    dest, vals = _gather_call(p2f, primals_3, primals_4, primals_5, primals_6)
    ipf = _scatter_call(dest, vals, p1f)
    ip = ipf.reshape(6, 256, 256)
    view = jnp.transpose(primals_8, (1, 0, 2))
    bmm6, pm6 = _tc_call(ip, primals_7, view)
    view_3 = jnp.transpose(bmm6, (1, 0, 2))
    return (view_3, pm6)


# speedup vs baseline: 3.0292x; 1.9534x over previous
import functools

import jax
import jax.numpy as jnp
from jax import lax
from jax.experimental import pallas as pl
from jax.experimental.pallas import tpu as pltpu
import jax.experimental.pallas.tpu_sc as plsc

N = 262144
NC = 2
NS = 16
NW = NC * NS
GCHUNK = N // NW
DEST = 6 * 256 * 256
OWN = DEST // NW
SCHUNK = 16384
NSCHUNK = N // SCHUNK

_mesh = plsc.VectorSubcoreMesh(
    core_axis_name="c", subcore_axis_name="s", num_cores=NC, num_subcores=NS
)
_sc_params = pltpu.CompilerParams(needs_layout_passes=False)


def _wid():
    return lax.axis_index("s") * NC + lax.axis_index("c")


def _gather_body(r0_hbm, r1_hbm, r2_hbm, r3_hbm, r4_hbm, r5_hbm,
                 p3_hbm, p4_hbm, p5_hbm, p6_hbm,
                 dest_hbm, vals_hbm,
                 i3, i4, i5, i6, dst, vals,
                 t0, t1, t2, t3, t4, t5, sem):
    base = _wid() * GCHUNK
    pltpu.sync_copy(p3_hbm.at[pl.ds(base, GCHUNK)], i3)
    pltpu.sync_copy(p4_hbm.at[pl.ds(base, GCHUNK)], i4)
    pltpu.sync_copy(p5_hbm.at[pl.ds(base, GCHUNK)], i5)
    pltpu.sync_copy(p6_hbm.at[pl.ds(base, GCHUNK)], i6)

    rows = (r0_hbm, r1_hbm, r2_hbm, r3_hbm, r4_hbm, r5_hbm)
    tmps = (t0, t1, t2, t3, t4, t5)
    for r in range(6):
        pltpu.async_copy(rows[r].at[i4], tmps[r], sem)

    def body(j, carry):
        sl = pl.ds(j * 16, 16)
        a3 = i3[sl]
        dst[sl] = a3 * 65536 + i5[sl] * 256 + i6[sl]
        return carry

    lax.fori_loop(0, GCHUNK // 16, body, 0, unroll=8)

    for r in range(6):
        pltpu.make_async_copy(rows[r].at[i4], tmps[r], sem).wait()

    def sel(j, carry):
        sl = pl.ds(j * 16, 16)
        a3 = i3[sl]
        v = t0[sl]
        for r in range(1, 6):
            v = jnp.where(a3 == r, tmps[r][sl], v)
        vals[sl] = v
        return carry

    lax.fori_loop(0, GCHUNK // 16, sel, 0, unroll=8)

    pltpu.sync_copy(dst, dest_hbm.at[pl.ds(base, GCHUNK)])
    pltpu.sync_copy(vals, vals_hbm.at[pl.ds(base, GCHUNK)])


_gather_call = pl.kernel(
    _gather_body,
    out_type=(
        jax.ShapeDtypeStruct((N,), jnp.int32),
        jax.ShapeDtypeStruct((N,), jnp.float32),
    ),
    mesh=_mesh,
    scratch_types=[
        pltpu.VMEM((GCHUNK,), jnp.int32),
        pltpu.VMEM((GCHUNK,), jnp.int32),
        pltpu.VMEM((GCHUNK,), jnp.int32),
        pltpu.VMEM((GCHUNK,), jnp.int32),
        pltpu.VMEM((GCHUNK,), jnp.int32),
        pltpu.VMEM((GCHUNK,), jnp.float32),
        pltpu.VMEM((GCHUNK,), jnp.float32),
        pltpu.VMEM((GCHUNK,), jnp.float32),
        pltpu.VMEM((GCHUNK,), jnp.float32),
        pltpu.VMEM((GCHUNK,), jnp.float32),
        pltpu.VMEM((GCHUNK,), jnp.float32),
        pltpu.VMEM((GCHUNK,), jnp.float32),
        pltpu.SemaphoreType.DMA,
    ],
    compiler_params=_sc_params,
)


def _scatter_body(dest_hbm, vals_hbm, p1_hbm, out_hbm, local,
                  dbuf0, dbuf1, vbuf0, vbuf1, semd, semv):
    wid = _wid()
    lo = wid * OWN
    pltpu.sync_copy(p1_hbm.at[pl.ds(lo, OWN)], local)
    lo_v = jnp.full((16,), 0, jnp.int32) + lo
    own_v = jnp.full((16,), OWN, jnp.uint32)

    dbufs = (dbuf0, dbuf1)
    vbufs = (vbuf0, vbuf1)

    def start(c, b):
        pltpu.async_copy(dest_hbm.at[pl.ds(c * SCHUNK, SCHUNK)], dbufs[b], semd)
        pltpu.async_copy(vals_hbm.at[pl.ds(c * SCHUNK, SCHUNK)], vbufs[b], semv)

    def wait(c, b):
        pltpu.make_async_copy(dest_hbm.at[pl.ds(c * SCHUNK, SCHUNK)],
                              dbufs[b], semd).wait()
        pltpu.make_async_copy(vals_hbm.at[pl.ds(c * SCHUNK, SCHUNK)],
                              vbufs[b], semv).wait()

    start(0, 0)
    for c in range(NSCHUNK):
        b = c % 2
        wait(c, b)
        if c + 1 < NSCHUNK:
            start(c + 1, 1 - b)
        dbuf = dbufs[b]
        vbuf = vbufs[b]

        def inner(j, carry):
            sl = pl.ds(j * 16, 16)
            u = plsc.bitcast(dbuf[sl] - lo_v, jnp.uint32)
            m = u < own_v
            plsc.store_scatter(local, [plsc.bitcast(u, jnp.int32)],
                               vbuf[sl], mask=m)
            return carry

        lax.fori_loop(0, SCHUNK // 16, inner, 0, unroll=8)

    pltpu.sync_copy(local, out_hbm.at[pl.ds(lo, OWN)])


_scatter_call = pl.kernel(
    _scatter_body,
    out_type=jax.ShapeDtypeStruct((DEST,), jnp.float32),
    mesh=_mesh,
    scratch_types=[
        pltpu.VMEM((OWN,), jnp.float32),
        pltpu.VMEM((SCHUNK,), jnp.int32),
        pltpu.VMEM((SCHUNK,), jnp.int32),
        pltpu.VMEM((SCHUNK,), jnp.float32),
        pltpu.VMEM((SCHUNK,), jnp.float32),
        pltpu.SemaphoreType.DMA,
        pltpu.SemaphoreType.DMA,
    ],
    compiler_params=_sc_params,
)


def _tc_body(ip_ref, p7_ref, v_ref, bmm_ref, pm6_ref):
    add = ip_ref[0] + p7_ref[0] * 0.975
    bmm_ref[0] = jnp.dot(v_ref[0], add, preferred_element_type=jnp.float32)
    pm6_ref[0] = add.T


_tc_call = pl.pallas_call(
    _tc_body,
    grid=(6,),
    in_specs=[
        pl.BlockSpec((1, 256, 256), lambda b: (b, 0, 0)),
        pl.BlockSpec((1, 256, 256), lambda b: (b, 0, 0)),
        pl.BlockSpec((1, 12, 256), lambda b: (b, 0, 0)),
    ],
    out_specs=[
        pl.BlockSpec((1, 12, 256), lambda b: (b, 0, 0)),
        pl.BlockSpec((1, 256, 256), lambda b: (b, 0, 0)),
    ],
    out_shape=[
        jax.ShapeDtypeStruct((6, 12, 256), jnp.float32),
        jax.ShapeDtypeStruct((6, 256, 256), jnp.float32),
    ],
)


@jax.jit
def kernel(primals_1, primals_2, primals_3, primals_4, primals_5, primals_6,
           primals_7, primals_8):
    p1f = primals_1.reshape(-1)
    dest, vals = _gather_call(
        primals_2[0], primals_2[1], primals_2[2], primals_2[3],
        primals_2[4], primals_2[5],
        primals_3, primals_4, primals_5, primals_6)
    ipf = _scatter_call(dest, vals, p1f)
    ip = ipf.reshape(6, 256, 256)
    view = jnp.transpose(primals_8, (1, 0, 2))
    bmm6, pm6 = _tc_call(ip, primals_7, view)
    view_3 = jnp.transpose(bmm6, (1, 0, 2))
    return (view_3, pm6)


# speedup vs baseline: 5.0230x; 1.6582x over previous
import jax
import jax.numpy as jnp
from jax import lax
from jax.experimental import pallas as pl
from jax.experimental.pallas import tpu as pltpu
import jax.experimental.pallas.tpu_sc as plsc

N = 262144
NC = 2
NS = 16
NW = NC * NS
GCHUNK = N // NW
DEST = 6 * 256 * 256
OWN = DEST // NW
SCHUNK = 16384
NSCHUNK = N // SCHUNK
TBL = 6 * 1000000
LCH = 32256
LTC = 999936

_mesh = plsc.VectorSubcoreMesh(
    core_axis_name="c", subcore_axis_name="s", num_cores=NC, num_subcores=NS
)
_sc_params = pltpu.CompilerParams(needs_layout_passes=False)


def _wid():
    return lax.axis_index("s") * NC + lax.axis_index("c")


def _prep_body(p2_hbm, p3_hbm, p4_hbm, p5_hbm, p6_hbm,
               p2lin_hbm, dest_hbm, lin_hbm,
               i3, i4, i5, i6, dst, lin, b0, b1, tbuf, sem):
    wid = _wid()
    base = wid * GCHUNK
    bufs = (b0, b1)

    @pl.when(wid < 31)
    def _():
        for r in range(6):
            buf = bufs[r % 2]
            src = p2_hbm.at[r, pl.ds(wid * LCH, LCH)]
            out = p2lin_hbm.at[pl.ds(r * 1000000 + wid * LCH, LCH)]
            if r >= 2:
                prev = p2lin_hbm.at[pl.ds((r - 2) * 1000000 + wid * LCH, LCH)]
                pltpu.make_async_copy(buf, prev, sem).wait()
            pltpu.sync_copy(src, buf)
            pltpu.async_copy(buf, out, sem)
        for r in range(4, 6):
            out = p2lin_hbm.at[pl.ds(r * 1000000 + wid * LCH, LCH)]
            pltpu.make_async_copy(bufs[r % 2], out, sem).wait()

    @pl.when(wid == 31)
    def _():
        pltpu.sync_copy(p2_hbm.at[pl.ds(0, 6), pl.ds(LTC, 64)], tbuf)
        for r in range(6):
            pltpu.sync_copy(tbuf.at[r],
                            p2lin_hbm.at[pl.ds(r * 1000000 + LTC, 64)])

    pltpu.sync_copy(p3_hbm.at[pl.ds(base, GCHUNK)], i3)
    pltpu.sync_copy(p4_hbm.at[pl.ds(base, GCHUNK)], i4)
    pltpu.sync_copy(p5_hbm.at[pl.ds(base, GCHUNK)], i5)
    pltpu.sync_copy(p6_hbm.at[pl.ds(base, GCHUNK)], i6)

    def body(j, carry):
        sl = pl.ds(j * 16, 16)
        a3 = i3[sl]
        lin[sl] = a3 * 1000000 + i4[sl]
        dst[sl] = a3 * 65536 + i5[sl] * 256 + i6[sl]
        return carry

    lax.fori_loop(0, GCHUNK // 16, body, 0, unroll=8)
    pltpu.sync_copy(dst, dest_hbm.at[pl.ds(base, GCHUNK)])
    pltpu.sync_copy(lin, lin_hbm.at[pl.ds(base, GCHUNK)])


_prep_call = pl.kernel(
    _prep_body,
    out_type=(
        jax.ShapeDtypeStruct((TBL,), jnp.float32),
        jax.ShapeDtypeStruct((N,), jnp.int32),
        jax.ShapeDtypeStruct((N,), jnp.int32),
    ),
    mesh=_mesh,
    scratch_types=[
        pltpu.VMEM((GCHUNK,), jnp.int32),
        pltpu.VMEM((GCHUNK,), jnp.int32),
        pltpu.VMEM((GCHUNK,), jnp.int32),
        pltpu.VMEM((GCHUNK,), jnp.int32),
        pltpu.VMEM((GCHUNK,), jnp.int32),
        pltpu.VMEM((GCHUNK,), jnp.int32),
        pltpu.VMEM((LCH,), jnp.float32),
        pltpu.VMEM((LCH,), jnp.float32),
        pltpu.VMEM((6, 64), jnp.float32),
        pltpu.SemaphoreType.DMA,
    ],
    compiler_params=_sc_params,
)


def _gather_body(p2lin_hbm, lin_hbm, vals_hbm, linv, vals, sem):
    base = _wid() * GCHUNK
    pltpu.sync_copy(lin_hbm.at[pl.ds(base, GCHUNK)], linv)
    pltpu.async_copy(p2lin_hbm.at[linv], vals, sem).wait()
    pltpu.sync_copy(vals, vals_hbm.at[pl.ds(base, GCHUNK)])


_gather_call = pl.kernel(
    _gather_body,
    out_type=jax.ShapeDtypeStruct((N,), jnp.float32),
    mesh=_mesh,
    scratch_types=[
        pltpu.VMEM((GCHUNK,), jnp.int32),
        pltpu.VMEM((GCHUNK,), jnp.float32),
        pltpu.SemaphoreType.DMA,
    ],
    compiler_params=_sc_params,
)


def _scatter_body(dest_hbm, vals_hbm, p1_hbm, out_hbm, local,
                  dbuf0, dbuf1, vbuf0, vbuf1, semd, semv):
    wid = _wid()
    lo = wid * OWN
    pltpu.sync_copy(p1_hbm.at[pl.ds(lo, OWN)], local.at[pl.ds(0, OWN)])
    lo_v = jnp.full((16,), 0, jnp.int32) + lo
    own_v = jnp.full((16,), OWN, jnp.uint32)

    dbufs = (dbuf0, dbuf1)
    vbufs = (vbuf0, vbuf1)

    def start(c, b):
        pltpu.async_copy(dest_hbm.at[pl.ds(c * SCHUNK, SCHUNK)], dbufs[b], semd)
        pltpu.async_copy(vals_hbm.at[pl.ds(c * SCHUNK, SCHUNK)], vbufs[b], semv)

    def wait(c, b):
        pltpu.make_async_copy(dest_hbm.at[pl.ds(c * SCHUNK, SCHUNK)],
                              dbufs[b], semd).wait()
        pltpu.make_async_copy(vals_hbm.at[pl.ds(c * SCHUNK, SCHUNK)],
                              vbufs[b], semv).wait()

    start(0, 0)
    for c in range(NSCHUNK):
        b = c % 2
        wait(c, b)
        if c + 1 < NSCHUNK:
            start(c + 1, 1 - b)
        dbuf = dbufs[b]
        vbuf = vbufs[b]

        def inner(j, carry):
            sl = pl.ds(j * 16, 16)
            u = plsc.bitcast(dbuf[sl] - lo_v, jnp.uint32)
            u = jnp.minimum(u, own_v)
            plsc.store_scatter(local, [plsc.bitcast(u, jnp.int32)], vbuf[sl])
            return carry

        lax.fori_loop(0, SCHUNK // 16, inner, 0, unroll=8)

    pltpu.sync_copy(local.at[pl.ds(0, OWN)], out_hbm.at[pl.ds(lo, OWN)])


_scatter_call = pl.kernel(
    _scatter_body,
    out_type=jax.ShapeDtypeStruct((DEST,), jnp.float32),
    mesh=_mesh,
    scratch_types=[
        pltpu.VMEM((OWN + 16,), jnp.float32),
        pltpu.VMEM((SCHUNK,), jnp.int32),
        pltpu.VMEM((SCHUNK,), jnp.int32),
        pltpu.VMEM((SCHUNK,), jnp.float32),
        pltpu.VMEM((SCHUNK,), jnp.float32),
        pltpu.SemaphoreType.DMA,
        pltpu.SemaphoreType.DMA,
    ],
    compiler_params=_sc_params,
)


def _tc_body(ip_ref, p7_ref, v_ref, bmm_ref, pm6_ref):
    add = ip_ref[0] + p7_ref[0] * 0.975
    bmm_ref[0] = jnp.dot(v_ref[0], add, preferred_element_type=jnp.float32)
    pm6_ref[0] = add.T


_tc_call = pl.pallas_call(
    _tc_body,
    grid=(6,),
    in_specs=[
        pl.BlockSpec((1, 256, 256), lambda b: (b, 0, 0)),
        pl.BlockSpec((1, 256, 256), lambda b: (b, 0, 0)),
        pl.BlockSpec((1, 12, 256), lambda b: (b, 0, 0)),
    ],
    out_specs=[
        pl.BlockSpec((1, 12, 256), lambda b: (b, 0, 0)),
        pl.BlockSpec((1, 256, 256), lambda b: (b, 0, 0)),
    ],
    out_shape=[
        jax.ShapeDtypeStruct((6, 12, 256), jnp.float32),
        jax.ShapeDtypeStruct((6, 256, 256), jnp.float32),
    ],
)


@jax.jit
def kernel(primals_1, primals_2, primals_3, primals_4, primals_5, primals_6,
           primals_7, primals_8):
    p1f = primals_1.reshape(-1)
    p2lin, dest, lin = _prep_call(primals_2, primals_3, primals_4,
                                  primals_5, primals_6)
    vals = _gather_call(p2lin, lin)
    ipf = _scatter_call(dest, vals, p1f)
    ip = ipf.reshape(6, 256, 256)
    view = jnp.transpose(primals_8, (1, 0, 2))
    bmm6, pm6 = _tc_call(ip, primals_7, view)
    view_3 = jnp.transpose(bmm6, (1, 0, 2))
    return (view_3, pm6)


# speedup vs baseline: 5.0754x; 1.0104x over previous
import jax
import jax.numpy as jnp
from jax import lax
from jax.experimental import pallas as pl
from jax.experimental.pallas import tpu as pltpu
import jax.experimental.pallas.tpu_sc as plsc

N = 262144
NC = 2
NS = 16
NW = NC * NS
GCHUNK = N // NW
DEST = 6 * 256 * 256
OWN = DEST // NW
SCHUNK = 16384
NSCHUNK = N // SCHUNK
TBL = 6 * 1000000
LCH = 32256
LTC = 999936

_mesh = plsc.VectorSubcoreMesh(
    core_axis_name="c", subcore_axis_name="s", num_cores=NC, num_subcores=NS
)
_sc_params = pltpu.CompilerParams(needs_layout_passes=False)


def _wid():
    return lax.axis_index("s") * NC + lax.axis_index("c")


def _prep_body(p2_hbm, p3_hbm, p4_hbm, p5_hbm, p6_hbm,
               p2lin_hbm, dest_hbm, lin_hbm,
               i3, i4, i5, i6, dst, lin, b0, b1, tbuf, sem):
    wid = _wid()
    base = wid * GCHUNK
    bufs = (b0, b1)

    @pl.when(wid < 31)
    def _():
        for r in range(6):
            buf = bufs[r % 2]
            src = p2_hbm.at[r, pl.ds(wid * LCH, LCH)]
            out = p2lin_hbm.at[pl.ds(r * 1000000 + wid * LCH, LCH)]
            if r >= 2:
                prev = p2lin_hbm.at[pl.ds((r - 2) * 1000000 + wid * LCH, LCH)]
                pltpu.make_async_copy(buf, prev, sem).wait()
            pltpu.sync_copy(src, buf)
            pltpu.async_copy(buf, out, sem)
        for r in range(4, 6):
            out = p2lin_hbm.at[pl.ds(r * 1000000 + wid * LCH, LCH)]
            pltpu.make_async_copy(bufs[r % 2], out, sem).wait()

    @pl.when(wid == 31)
    def _():
        pltpu.sync_copy(p2_hbm.at[pl.ds(0, 6), pl.ds(LTC, 64)], tbuf)
        for r in range(6):
            pltpu.sync_copy(tbuf.at[r],
                            p2lin_hbm.at[pl.ds(r * 1000000 + LTC, 64)])

    pltpu.sync_copy(p3_hbm.at[pl.ds(base, GCHUNK)], i3)
    pltpu.sync_copy(p4_hbm.at[pl.ds(base, GCHUNK)], i4)
    pltpu.sync_copy(p5_hbm.at[pl.ds(base, GCHUNK)], i5)
    pltpu.sync_copy(p6_hbm.at[pl.ds(base, GCHUNK)], i6)

    def body(j, carry):
        sl = pl.ds(j * 16, 16)
        a3 = i3[sl]
        lin[sl] = a3 * 1000000 + i4[sl]
        dst[sl] = a3 * 65536 + i5[sl] * 256 + i6[sl]
        return carry

    lax.fori_loop(0, GCHUNK // 16, body, 0, unroll=8)
    pltpu.sync_copy(dst, dest_hbm.at[pl.ds(base, GCHUNK)])
    pltpu.sync_copy(lin, lin_hbm.at[pl.ds(base, GCHUNK)])


_prep_call = pl.kernel(
    _prep_body,
    out_type=(
        jax.ShapeDtypeStruct((TBL,), jnp.float32),
        jax.ShapeDtypeStruct((N,), jnp.int32),
        jax.ShapeDtypeStruct((N,), jnp.int32),
    ),
    mesh=_mesh,
    scratch_types=[
        pltpu.VMEM((GCHUNK,), jnp.int32),
        pltpu.VMEM((GCHUNK,), jnp.int32),
        pltpu.VMEM((GCHUNK,), jnp.int32),
        pltpu.VMEM((GCHUNK,), jnp.int32),
        pltpu.VMEM((GCHUNK,), jnp.int32),
        pltpu.VMEM((GCHUNK,), jnp.int32),
        pltpu.VMEM((LCH,), jnp.float32),
        pltpu.VMEM((LCH,), jnp.float32),
        pltpu.VMEM((6, 64), jnp.float32),
        pltpu.SemaphoreType.DMA,
    ],
    compiler_params=_sc_params,
)


def _gather_body(p2lin_hbm, lin_hbm, vals_hbm, linv, vals, sem):
    base = _wid() * GCHUNK
    pltpu.sync_copy(lin_hbm.at[pl.ds(base, GCHUNK)], linv)
    pltpu.async_copy(p2lin_hbm.at[linv], vals, sem).wait()
    pltpu.sync_copy(vals, vals_hbm.at[pl.ds(base, GCHUNK)])


_gather_call = pl.kernel(
    _gather_body,
    out_type=jax.ShapeDtypeStruct((N,), jnp.float32),
    mesh=_mesh,
    scratch_types=[
        pltpu.VMEM((GCHUNK,), jnp.int32),
        pltpu.VMEM((GCHUNK,), jnp.float32),
        pltpu.SemaphoreType.DMA,
    ],
    compiler_params=_sc_params,
)


def _scatter_body(dest_hbm, vals_hbm, p1_hbm, out_hbm, local,
                  dbuf0, dbuf1, vbuf0, vbuf1, semd, semv):
    wid = _wid()
    lo = wid * OWN
    pltpu.sync_copy(p1_hbm.at[pl.ds(lo, OWN)], local.at[pl.ds(0, OWN)])
    lo_v = jnp.full((16,), 0, jnp.int32) + lo
    lane = lax.broadcasted_iota(jnp.int32, (16,), 0)
    own_v = plsc.bitcast(lane + OWN, jnp.uint32)

    dbufs = (dbuf0, dbuf1)
    vbufs = (vbuf0, vbuf1)

    def start(c, b):
        pltpu.async_copy(dest_hbm.at[pl.ds(c * SCHUNK, SCHUNK)], dbufs[b], semd)
        pltpu.async_copy(vals_hbm.at[pl.ds(c * SCHUNK, SCHUNK)], vbufs[b], semv)

    def wait(c, b):
        pltpu.make_async_copy(dest_hbm.at[pl.ds(c * SCHUNK, SCHUNK)],
                              dbufs[b], semd).wait()
        pltpu.make_async_copy(vals_hbm.at[pl.ds(c * SCHUNK, SCHUNK)],
                              vbufs[b], semv).wait()

    start(0, 0)
    for c in range(NSCHUNK):
        b = c % 2
        wait(c, b)
        if c + 1 < NSCHUNK:
            start(c + 1, 1 - b)
        dbuf = dbufs[b]
        vbuf = vbufs[b]

        def inner(j, carry):
            sl = pl.ds(j * 16, 16)
            u = plsc.bitcast(dbuf[sl] - lo_v, jnp.uint32)
            u = jnp.minimum(u, own_v)
            plsc.store_scatter(local, [plsc.bitcast(u, jnp.int32)], vbuf[sl])
            return carry

        lax.fori_loop(0, SCHUNK // 16, inner, 0, unroll=8)

    pltpu.sync_copy(local.at[pl.ds(0, OWN)], out_hbm.at[pl.ds(lo, OWN)])


_scatter_call = pl.kernel(
    _scatter_body,
    out_type=jax.ShapeDtypeStruct((DEST,), jnp.float32),
    mesh=_mesh,
    scratch_types=[
        pltpu.VMEM((OWN + 16,), jnp.float32),
        pltpu.VMEM((SCHUNK,), jnp.int32),
        pltpu.VMEM((SCHUNK,), jnp.int32),
        pltpu.VMEM((SCHUNK,), jnp.float32),
        pltpu.VMEM((SCHUNK,), jnp.float32),
        pltpu.SemaphoreType.DMA,
        pltpu.SemaphoreType.DMA,
    ],
    compiler_params=_sc_params,
)


def _tc_body(ip_ref, p7_ref, v_ref, bmm_ref, pm6_ref):
    add = ip_ref[0] + p7_ref[0] * 0.975
    bmm_ref[0] = jnp.dot(v_ref[0], add, preferred_element_type=jnp.float32)
    pm6_ref[0] = add.T


_tc_call = pl.pallas_call(
    _tc_body,
    grid=(6,),
    in_specs=[
        pl.BlockSpec((1, 256, 256), lambda b: (b, 0, 0)),
        pl.BlockSpec((1, 256, 256), lambda b: (b, 0, 0)),
        pl.BlockSpec((1, 12, 256), lambda b: (b, 0, 0)),
    ],
    out_specs=[
        pl.BlockSpec((1, 12, 256), lambda b: (b, 0, 0)),
        pl.BlockSpec((1, 256, 256), lambda b: (b, 0, 0)),
    ],
    out_shape=[
        jax.ShapeDtypeStruct((6, 12, 256), jnp.float32),
        jax.ShapeDtypeStruct((6, 256, 256), jnp.float32),
    ],
)


@jax.jit
def kernel(primals_1, primals_2, primals_3, primals_4, primals_5, primals_6,
           primals_7, primals_8):
    p1f = primals_1.reshape(-1)
    p2lin, dest, lin = _prep_call(primals_2, primals_3, primals_4,
                                  primals_5, primals_6)
    vals = _gather_call(p2lin, lin)
    ipf = _scatter_call(dest, vals, p1f)
    ip = ipf.reshape(6, 256, 256)
    view = jnp.transpose(primals_8, (1, 0, 2))
    bmm6, pm6 = _tc_call(ip, primals_7, view)
    view_3 = jnp.transpose(bmm6, (1, 0, 2))
    return (view_3, pm6)


# speedup vs baseline: 7.8947x; 1.5555x over previous
import jax
import jax.numpy as jnp
from jax import lax
from jax.experimental import pallas as pl
from jax.experimental.pallas import tpu as pltpu
import jax.experimental.pallas.tpu_sc as plsc

N = 262144
NC = 2
NS = 16
NW = NC * NS
GCHUNK = N // NW
DEST = 6 * 256 * 256
OWN = DEST // NW
SCHUNK = 16384
NSCHUNK = N // SCHUNK
TBL = 6 * 1000000
LCH = 32256
LTC = 999936

_mesh = plsc.VectorSubcoreMesh(
    core_axis_name="c", subcore_axis_name="s", num_cores=NC, num_subcores=NS
)
_sc_params = pltpu.CompilerParams(needs_layout_passes=False)


def _wid():
    return lax.axis_index("s") * NC + lax.axis_index("c")


def _prep_body(p2_hbm, p3_hbm, p4_hbm, p5_hbm, p6_hbm,
               p2lin_hbm, dest_hbm, lin_hbm,
               i3, i4, i5, i6, dst, lin, b0, b1, tbuf, sem):
    wid = _wid()
    base = wid * GCHUNK
    bufs = (b0, b1)

    @pl.when(wid < 31)
    def _():
        for r in range(6):
            buf = bufs[r % 2]
            src = p2_hbm.at[r, pl.ds(wid * LCH, LCH)]
            out = p2lin_hbm.at[pl.ds(r * 1000000 + wid * LCH, LCH)]
            if r >= 2:
                prev = p2lin_hbm.at[pl.ds((r - 2) * 1000000 + wid * LCH, LCH)]
                pltpu.make_async_copy(buf, prev, sem).wait()
            pltpu.sync_copy(src, buf)
            pltpu.async_copy(buf, out, sem)
        for r in range(4, 6):
            out = p2lin_hbm.at[pl.ds(r * 1000000 + wid * LCH, LCH)]
            pltpu.make_async_copy(bufs[r % 2], out, sem).wait()

    @pl.when(wid == 31)
    def _():
        pltpu.sync_copy(p2_hbm.at[pl.ds(0, 6), pl.ds(LTC, 64)], tbuf)
        for r in range(6):
            pltpu.sync_copy(tbuf.at[r],
                            p2lin_hbm.at[pl.ds(r * 1000000 + LTC, 64)])

    pltpu.sync_copy(p3_hbm.at[pl.ds(base, GCHUNK)], i3)
    pltpu.sync_copy(p4_hbm.at[pl.ds(base, GCHUNK)], i4)
    pltpu.sync_copy(p5_hbm.at[pl.ds(base, GCHUNK)], i5)
    pltpu.sync_copy(p6_hbm.at[pl.ds(base, GCHUNK)], i6)

    def body(j, carry):
        sl = pl.ds(j * 16, 16)
        a3 = i3[sl]
        lin[sl] = a3 * 1000000 + i4[sl]
        dst[sl] = a3 * 65536 + i5[sl] * 256 + i6[sl]
        return carry

    lax.fori_loop(0, GCHUNK // 16, body, 0, unroll=8)
    pltpu.sync_copy(dst, dest_hbm.at[pl.ds(base, GCHUNK)])
    pltpu.sync_copy(lin, lin_hbm.at[pl.ds(base, GCHUNK)])


_prep_call = pl.kernel(
    _prep_body,
    out_type=(
        jax.ShapeDtypeStruct((TBL,), jnp.float32),
        jax.ShapeDtypeStruct((N,), jnp.int32),
        jax.ShapeDtypeStruct((N,), jnp.int32),
    ),
    mesh=_mesh,
    scratch_types=[
        pltpu.VMEM((GCHUNK,), jnp.int32),
        pltpu.VMEM((GCHUNK,), jnp.int32),
        pltpu.VMEM((GCHUNK,), jnp.int32),
        pltpu.VMEM((GCHUNK,), jnp.int32),
        pltpu.VMEM((GCHUNK,), jnp.int32),
        pltpu.VMEM((GCHUNK,), jnp.int32),
        pltpu.VMEM((LCH,), jnp.float32),
        pltpu.VMEM((LCH,), jnp.float32),
        pltpu.VMEM((6, 64), jnp.float32),
        pltpu.SemaphoreType.DMA,
    ],
    compiler_params=_sc_params,
)


def _gather_body(p2lin_hbm, lin_hbm, vals_hbm, linv, vals, sem):
    base = _wid() * GCHUNK
    pltpu.sync_copy(lin_hbm.at[pl.ds(base, GCHUNK)], linv)
    pltpu.async_copy(p2lin_hbm.at[linv], vals, sem).wait()
    pltpu.sync_copy(vals, vals_hbm.at[pl.ds(base, GCHUNK)])


_gather_call = pl.kernel(
    _gather_body,
    out_type=jax.ShapeDtypeStruct((N,), jnp.float32),
    mesh=_mesh,
    scratch_types=[
        pltpu.VMEM((GCHUNK,), jnp.int32),
        pltpu.VMEM((GCHUNK,), jnp.float32),
        pltpu.SemaphoreType.DMA,
    ],
    compiler_params=_sc_params,
)


def _scatter_body(dest_hbm, vals_hbm, p1_hbm, out_hbm, local,
                  dbuf0, dbuf1, vbuf0, vbuf1, semd, semv):
    wid = _wid()
    lo = wid * OWN
    pltpu.sync_copy(p1_hbm.at[pl.ds(lo, OWN)], local.at[pl.ds(0, OWN)])
    lo_v = jnp.full((16,), 0, jnp.int32) + lo
    lane = lax.broadcasted_iota(jnp.int32, (16,), 0)
    own_v = plsc.bitcast(lane + OWN, jnp.uint32)

    dbufs = (dbuf0, dbuf1)
    vbufs = (vbuf0, vbuf1)

    def start(c, b):
        pltpu.async_copy(dest_hbm.at[pl.ds(c * SCHUNK, SCHUNK)], dbufs[b], semd)
        pltpu.async_copy(vals_hbm.at[pl.ds(c * SCHUNK, SCHUNK)], vbufs[b], semv)

    def wait(c, b):
        pltpu.make_async_copy(dest_hbm.at[pl.ds(c * SCHUNK, SCHUNK)],
                              dbufs[b], semd).wait()
        pltpu.make_async_copy(vals_hbm.at[pl.ds(c * SCHUNK, SCHUNK)],
                              vbufs[b], semv).wait()

    start(0, 0)
    for c in range(NSCHUNK):
        b = c % 2
        wait(c, b)
        if c + 1 < NSCHUNK:
            start(c + 1, 1 - b)
        dbuf = dbufs[b]
        vbuf = vbufs[b]

        def inner(j, carry):
            base_j = j * 128
            us = []
            for k in range(8):
                sl = pl.ds(base_j + k * 16, 16)
                u = plsc.bitcast(dbuf[sl] - lo_v, jnp.uint32)
                us.append(jnp.minimum(u, own_v))
            vs = [vbuf[pl.ds(base_j + k * 16, 16)] for k in range(8)]
            for k in range(8):
                plsc.store_scatter(local, [plsc.bitcast(us[k], jnp.int32)],
                                   vs[k])
            return carry

        lax.fori_loop(0, SCHUNK // 128, inner, 0)

    pltpu.sync_copy(local.at[pl.ds(0, OWN)], out_hbm.at[pl.ds(lo, OWN)])


_scatter_call = pl.kernel(
    _scatter_body,
    out_type=jax.ShapeDtypeStruct((DEST,), jnp.float32),
    mesh=_mesh,
    scratch_types=[
        pltpu.VMEM((OWN + 16,), jnp.float32),
        pltpu.VMEM((SCHUNK,), jnp.int32),
        pltpu.VMEM((SCHUNK,), jnp.int32),
        pltpu.VMEM((SCHUNK,), jnp.float32),
        pltpu.VMEM((SCHUNK,), jnp.float32),
        pltpu.SemaphoreType.DMA,
        pltpu.SemaphoreType.DMA,
    ],
    compiler_params=_sc_params,
)


def _tc_body(ip_ref, p7_ref, v_ref, bmm_ref, pm6_ref):
    add = ip_ref[0] + p7_ref[0] * 0.975
    bmm_ref[0] = jnp.dot(v_ref[0], add, preferred_element_type=jnp.float32)
    pm6_ref[0] = add.T


_tc_call = pl.pallas_call(
    _tc_body,
    grid=(6,),
    in_specs=[
        pl.BlockSpec((1, 256, 256), lambda b: (b, 0, 0)),
        pl.BlockSpec((1, 256, 256), lambda b: (b, 0, 0)),
        pl.BlockSpec((1, 12, 256), lambda b: (b, 0, 0)),
    ],
    out_specs=[
        pl.BlockSpec((1, 12, 256), lambda b: (b, 0, 0)),
        pl.BlockSpec((1, 256, 256), lambda b: (b, 0, 0)),
    ],
    out_shape=[
        jax.ShapeDtypeStruct((6, 12, 256), jnp.float32),
        jax.ShapeDtypeStruct((6, 256, 256), jnp.float32),
    ],
)


@jax.jit
def kernel(primals_1, primals_2, primals_3, primals_4, primals_5, primals_6,
           primals_7, primals_8):
    p1f = primals_1.reshape(-1)
    p2lin, dest, lin = _prep_call(primals_2, primals_3, primals_4,
                                  primals_5, primals_6)
    vals = _gather_call(p2lin, lin)
    ipf = _scatter_call(dest, vals, p1f)
    ip = ipf.reshape(6, 256, 256)
    view = jnp.transpose(primals_8, (1, 0, 2))
    bmm6, pm6 = _tc_call(ip, primals_7, view)
    view_3 = jnp.transpose(bmm6, (1, 0, 2))
    return (view_3, pm6)


# speedup vs baseline: 8.1554x; 1.0330x over previous
import jax
import jax.numpy as jnp
from jax import lax
from jax.experimental import pallas as pl
from jax.experimental.pallas import tpu as pltpu
import jax.experimental.pallas.tpu_sc as plsc

N = 262144
NC = 2
NS = 16
NW = NC * NS
GCHUNK = N // NW
DEST = 6 * 256 * 256
OWN = DEST // NW
SCHUNK = 16384
NSCHUNK = N // SCHUNK
TBL = 6 * 1000000
LCH = 32256
LTC = 999936

_mesh = plsc.VectorSubcoreMesh(
    core_axis_name="c", subcore_axis_name="s", num_cores=NC, num_subcores=NS
)
_sc_params = pltpu.CompilerParams(needs_layout_passes=False)


def _wid():
    return lax.axis_index("s") * NC + lax.axis_index("c")


def _prep_body(p2_hbm, p3_hbm, p4_hbm, p5_hbm, p6_hbm,
               p2lin_hbm, dest_hbm, lin_hbm,
               i3, i4, i5, i6, dst, lin, b0, b1, tbuf, sem):
    wid = _wid()
    base = wid * GCHUNK
    bufs = (b0, b1)

    @pl.when(wid < 31)
    def _():
        for r in range(6):
            buf = bufs[r % 2]
            src = p2_hbm.at[r, pl.ds(wid * LCH, LCH)]
            out = p2lin_hbm.at[pl.ds(r * 1000000 + wid * LCH, LCH)]
            if r >= 2:
                prev = p2lin_hbm.at[pl.ds((r - 2) * 1000000 + wid * LCH, LCH)]
                pltpu.make_async_copy(buf, prev, sem).wait()
            pltpu.sync_copy(src, buf)
            pltpu.async_copy(buf, out, sem)
        for r in range(4, 6):
            out = p2lin_hbm.at[pl.ds(r * 1000000 + wid * LCH, LCH)]
            pltpu.make_async_copy(bufs[r % 2], out, sem).wait()

    @pl.when(wid == 31)
    def _():
        pltpu.sync_copy(p2_hbm.at[pl.ds(0, 6), pl.ds(LTC, 64)], tbuf)
        for r in range(6):
            pltpu.sync_copy(tbuf.at[r],
                            p2lin_hbm.at[pl.ds(r * 1000000 + LTC, 64)])

    pltpu.sync_copy(p3_hbm.at[pl.ds(base, GCHUNK)], i3)
    pltpu.sync_copy(p4_hbm.at[pl.ds(base, GCHUNK)], i4)
    pltpu.sync_copy(p5_hbm.at[pl.ds(base, GCHUNK)], i5)
    pltpu.sync_copy(p6_hbm.at[pl.ds(base, GCHUNK)], i6)

    def body(j, carry):
        base_j = j * 128
        for k in range(8):
            sl = pl.ds(base_j + k * 16, 16)
            a3 = i3[sl]
            lin[sl] = a3 * 1000000 + i4[sl]
            dst[sl] = a3 * 65536 + i5[sl] * 256 + i6[sl]
        return carry

    lax.fori_loop(0, GCHUNK // 128, body, 0)
    pltpu.sync_copy(dst, dest_hbm.at[pl.ds(base, GCHUNK)])
    pltpu.sync_copy(lin, lin_hbm.at[pl.ds(base, GCHUNK)])


_prep_call = pl.kernel(
    _prep_body,
    out_type=(
        jax.ShapeDtypeStruct((TBL,), jnp.float32),
        jax.ShapeDtypeStruct((N,), jnp.int32),
        jax.ShapeDtypeStruct((N,), jnp.int32),
    ),
    mesh=_mesh,
    scratch_types=[
        pltpu.VMEM((GCHUNK,), jnp.int32),
        pltpu.VMEM((GCHUNK,), jnp.int32),
        pltpu.VMEM((GCHUNK,), jnp.int32),
        pltpu.VMEM((GCHUNK,), jnp.int32),
        pltpu.VMEM((GCHUNK,), jnp.int32),
        pltpu.VMEM((GCHUNK,), jnp.int32),
        pltpu.VMEM((LCH,), jnp.float32),
        pltpu.VMEM((LCH,), jnp.float32),
        pltpu.VMEM((6, 64), jnp.float32),
        pltpu.SemaphoreType.DMA,
    ],
    compiler_params=_sc_params,
)


def _gather_body(p2lin_hbm, lin_hbm, vals_hbm, linv, vals, sem):
    base = _wid() * GCHUNK
    pltpu.sync_copy(lin_hbm.at[pl.ds(base, GCHUNK)], linv)
    pltpu.async_copy(p2lin_hbm.at[linv], vals, sem).wait()
    pltpu.sync_copy(vals, vals_hbm.at[pl.ds(base, GCHUNK)])


_gather_call = pl.kernel(
    _gather_body,
    out_type=jax.ShapeDtypeStruct((N,), jnp.float32),
    mesh=_mesh,
    scratch_types=[
        pltpu.VMEM((GCHUNK,), jnp.int32),
        pltpu.VMEM((GCHUNK,), jnp.float32),
        pltpu.SemaphoreType.DMA,
    ],
    compiler_params=_sc_params,
)


def _scatter_body(dest_hbm, vals_hbm, p1_hbm, out_hbm, local,
                  dbuf0, dbuf1, vbuf0, vbuf1, semd, semv):
    wid = _wid()
    lo = wid * OWN
    pltpu.sync_copy(p1_hbm.at[pl.ds(lo, OWN)], local.at[pl.ds(0, OWN)])
    lo_v = jnp.full((16,), 0, jnp.int32) + lo
    lane = lax.broadcasted_iota(jnp.int32, (16,), 0)
    own_v = plsc.bitcast(lane + OWN, jnp.uint32)

    dbufs = (dbuf0, dbuf1)
    vbufs = (vbuf0, vbuf1)

    def start(c, b):
        pltpu.async_copy(dest_hbm.at[pl.ds(c * SCHUNK, SCHUNK)], dbufs[b], semd)
        pltpu.async_copy(vals_hbm.at[pl.ds(c * SCHUNK, SCHUNK)], vbufs[b], semv)

    def wait(c, b):
        pltpu.make_async_copy(dest_hbm.at[pl.ds(c * SCHUNK, SCHUNK)],
                              dbufs[b], semd).wait()
        pltpu.make_async_copy(vals_hbm.at[pl.ds(c * SCHUNK, SCHUNK)],
                              vbufs[b], semv).wait()

    start(0, 0)
    for c in range(NSCHUNK):
        b = c % 2
        wait(c, b)
        if c + 1 < NSCHUNK:
            start(c + 1, 1 - b)
        dbuf = dbufs[b]
        vbuf = vbufs[b]

        def inner(j, carry):
            base_j = j * 128
            us = []
            for k in range(8):
                sl = pl.ds(base_j + k * 16, 16)
                u = plsc.bitcast(dbuf[sl] - lo_v, jnp.uint32)
                us.append(jnp.minimum(u, own_v))
            vs = [vbuf[pl.ds(base_j + k * 16, 16)] for k in range(8)]
            for k in range(8):
                plsc.store_scatter(local, [plsc.bitcast(us[k], jnp.int32)],
                                   vs[k])
            return carry

        lax.fori_loop(0, SCHUNK // 128, inner, 0)

    pltpu.sync_copy(local.at[pl.ds(0, OWN)], out_hbm.at[pl.ds(lo, OWN)])


_scatter_call = pl.kernel(
    _scatter_body,
    out_type=jax.ShapeDtypeStruct((DEST,), jnp.float32),
    mesh=_mesh,
    scratch_types=[
        pltpu.VMEM((OWN + 16,), jnp.float32),
        pltpu.VMEM((SCHUNK,), jnp.int32),
        pltpu.VMEM((SCHUNK,), jnp.int32),
        pltpu.VMEM((SCHUNK,), jnp.float32),
        pltpu.VMEM((SCHUNK,), jnp.float32),
        pltpu.SemaphoreType.DMA,
        pltpu.SemaphoreType.DMA,
    ],
    compiler_params=_sc_params,
)


def _tc_body(ip_ref, p7_ref, v_ref, bmm_ref, pm6_ref):
    add = ip_ref[0] + p7_ref[0] * 0.975
    bmm_ref[0] = jnp.dot(v_ref[0], add, preferred_element_type=jnp.float32)
    pm6_ref[0] = add.T


_tc_call = pl.pallas_call(
    _tc_body,
    grid=(6,),
    in_specs=[
        pl.BlockSpec((1, 256, 256), lambda b: (b, 0, 0)),
        pl.BlockSpec((1, 256, 256), lambda b: (b, 0, 0)),
        pl.BlockSpec((1, 12, 256), lambda b: (b, 0, 0)),
    ],
    out_specs=[
        pl.BlockSpec((1, 12, 256), lambda b: (b, 0, 0)),
        pl.BlockSpec((1, 256, 256), lambda b: (b, 0, 0)),
    ],
    out_shape=[
        jax.ShapeDtypeStruct((6, 12, 256), jnp.float32),
        jax.ShapeDtypeStruct((6, 256, 256), jnp.float32),
    ],
)


@jax.jit
def kernel(primals_1, primals_2, primals_3, primals_4, primals_5, primals_6,
           primals_7, primals_8):
    p1f = primals_1.reshape(-1)
    p2lin, dest, lin = _prep_call(primals_2, primals_3, primals_4,
                                  primals_5, primals_6)
    vals = _gather_call(p2lin, lin)
    ipf = _scatter_call(dest, vals, p1f)
    ip = ipf.reshape(6, 256, 256)
    view = jnp.transpose(primals_8, (1, 0, 2))
    bmm6, pm6 = _tc_call(ip, primals_7, view)
    view_3 = jnp.transpose(bmm6, (1, 0, 2))
    return (view_3, pm6)
